# baseline (speedup 1.0000x reference)
import functools

import jax
import jax.numpy as jnp
from jax import lax
from jax.experimental import pallas as pl
from jax.experimental.pallas import tpu as pltpu
from jax.experimental.pallas import tpu_sc as plsc

_NUM_ELEMENTS = 119
_FEAT_DIM = 92
_D_MODEL = 256
_N_ATOMS = 100000

_VPAD = 128
_FPAD = 128

_NC = 2
_NS = 16
_NW = _NC * _NS

_CHUNK = 80
_NUM_CHUNKS = _N_ATOMS // _CHUNK


def _fuse_body(t_ref, w_ref, b_ref, o_ref):
    o_ref[...] = (
        jnp.dot(t_ref[...], w_ref[...], preferred_element_type=jnp.float32)
        + b_ref[...]
    )


def _fused_table(table, W, b):
    tp = jnp.zeros((_VPAD, _FPAD), jnp.float32).at[:_NUM_ELEMENTS, :_FEAT_DIM].set(table)
    wp = jnp.zeros((_FPAD, _D_MODEL), jnp.float32).at[:_FEAT_DIM].set(W)
    return pl.pallas_call(
        _fuse_body,
        out_shape=jax.ShapeDtypeStruct((_VPAD, _D_MODEL), jnp.float32),
    )(tp, wp, b.reshape(1, _D_MODEL))


_mesh = plsc.VectorSubcoreMesh(
    core_axis_name="c", subcore_axis_name="s", num_cores=_NC, num_subcores=_NS
)


@functools.partial(
    pl.kernel,
    out_type=jax.ShapeDtypeStruct((_N_ATOMS, _D_MODEL), jnp.float32),
    mesh=_mesh,
    scratch_types=[
        pltpu.VMEM((_CHUNK,), jnp.int32),
        pltpu.VMEM((_CHUNK, _D_MODEL), jnp.float32),
        pltpu.SemaphoreType.DMA,
    ],
)
def _gather(x_hbm, fused_hbm, out_hbm, idx_v, rows_v, sem):
    wid = lax.axis_index("s") * _NC + lax.axis_index("c")
    n_lo = _NUM_CHUNKS // _NW
    n_mine = jnp.where(wid < _NUM_CHUNKS - n_lo * _NW, n_lo + 1, n_lo)

    def body(i, carry):
        base = (wid + i * _NW) * _CHUNK
        pltpu.sync_copy(x_hbm.at[pl.ds(base, _CHUNK)], idx_v)
        pltpu.async_copy(fused_hbm.at[idx_v], rows_v, sem).wait()
        pltpu.sync_copy(rows_v, out_hbm.at[pl.ds(base, _CHUNK)])
        return carry

    lax.fori_loop(0, n_mine, body, 0)


def kernel(x, table, W, b):
    fused = _fused_table(table, W, b)
    return _gather(x, fused)


# speedup vs baseline: 1.0015x; 1.0015x over previous
import functools

import jax
import jax.numpy as jnp
from jax import lax
from jax.experimental import pallas as pl
from jax.experimental.pallas import tpu as pltpu
from jax.experimental.pallas import tpu_sc as plsc

_NUM_ELEMENTS = 119
_FEAT_DIM = 92
_D_MODEL = 256
_N_ATOMS = 100000

_VPAD = 128
_FPAD = 128

_NC = 2
_NS = 16
_NW = _NC * _NS

_CHUNK = 80
_NUM_CHUNKS = _N_ATOMS // _CHUNK
_NBUF = 2
_NI = -(-_NUM_CHUNKS // _NW)


def _fuse_body(t_ref, w_ref, b_ref, o_ref):
    o_ref[...] = (
        jnp.dot(t_ref[...], w_ref[...], preferred_element_type=jnp.float32)
        + b_ref[...]
    )


def _fused_table(table, W, b):
    tp = jnp.zeros((_VPAD, _FPAD), jnp.float32).at[:_NUM_ELEMENTS, :_FEAT_DIM].set(table)
    wp = jnp.zeros((_FPAD, _D_MODEL), jnp.float32).at[:_FEAT_DIM].set(W)
    return pl.pallas_call(
        _fuse_body,
        out_shape=jax.ShapeDtypeStruct((_VPAD, _D_MODEL), jnp.float32),
    )(tp, wp, b.reshape(1, _D_MODEL))


_mesh = plsc.VectorSubcoreMesh(
    core_axis_name="c", subcore_axis_name="s", num_cores=_NC, num_subcores=_NS
)


@functools.partial(
    pl.kernel,
    out_type=jax.ShapeDtypeStruct((_N_ATOMS, _D_MODEL), jnp.float32),
    mesh=_mesh,
    scratch_types=[
        pltpu.VMEM((_NBUF, _CHUNK), jnp.int32),
        pltpu.VMEM((_NBUF, _CHUNK, _D_MODEL), jnp.float32),
        pltpu.SemaphoreType.DMA,
        pltpu.SemaphoreType.DMA,
        pltpu.SemaphoreType.DMA,
        pltpu.SemaphoreType.DMA,
        pltpu.SemaphoreType.DMA,
    ],
)
def _gather(x_hbm, fused_hbm, out_hbm, idx_v, rows_v, gsem, isem0, isem1, wsem0, wsem1):
    wid = lax.axis_index("s") * _NC + lax.axis_index("c")
    isems = (isem0, isem1)
    wsems = (wsem0, wsem1)

    def cid(i):
        return wid + i * _NW

    def start_idx(i, p):
        pltpu.async_copy(
            x_hbm.at[pl.ds(cid(i) * _CHUNK, _CHUNK)], idx_v.at[p], isems[p]
        )

    for p in range(_NBUF):
        start_idx(p, p)

    def body(k, carry):
        for p in range(_NBUF):
            i = _NBUF * k + p

            @pl.when(cid(i) < _NUM_CHUNKS)
            def _process():
                pltpu.make_async_copy(
                    x_hbm.at[pl.ds(0, _CHUNK)], idx_v.at[p], isems[p]
                ).wait()

                @pl.when(k >= 1)
                def _drain_prev():
                    pltpu.make_async_copy(
                        rows_v.at[p], out_hbm.at[pl.ds(0, _CHUNK)], wsems[p]
                    ).wait()

                pltpu.async_copy(fused_hbm.at[idx_v.at[p]], rows_v.at[p], gsem).wait()

                pltpu.async_copy(
                    rows_v.at[p], out_hbm.at[pl.ds(cid(i) * _CHUNK, _CHUNK)], wsems[p]
                )

                @pl.when(cid(i + _NBUF) < _NUM_CHUNKS)
                def _prefetch():
                    start_idx(i + _NBUF, p)

        return carry

    lax.fori_loop(0, _NI // _NBUF, body, 0)

    for p in range(_NBUF):
        pltpu.make_async_copy(
            rows_v.at[p], out_hbm.at[pl.ds(0, _CHUNK)], wsems[p]
        ).wait()


def kernel(x, table, W, b):
    fused = _fused_table(table, W, b)
    return _gather(x, fused)


# speedup vs baseline: 1.0064x; 1.0049x over previous
import functools

import jax
import jax.numpy as jnp
from jax import lax
from jax.experimental import pallas as pl
from jax.experimental.pallas import tpu as pltpu
from jax.experimental.pallas import tpu_sc as plsc

_NUM_ELEMENTS = 119
_FEAT_DIM = 92
_D_MODEL = 256
_N_ATOMS = 100000

_VPAD = 128
_FPAD = 128

_NC = 2
_NS = 16
_NW = _NC * _NS

_CHUNK = 80
_NUM_CHUNKS = _N_ATOMS // _CHUNK
_NBUF = 4
_NI = -(-_NUM_CHUNKS // _NW)
assert _NI % _NBUF == 0


def _fuse_body(t_ref, w_ref, b_ref, o_ref):
    o_ref[...] = (
        jnp.dot(t_ref[...], w_ref[...], preferred_element_type=jnp.float32)
        + b_ref[...]
    )


def _fused_table(table, W, b):
    tp = jnp.zeros((_VPAD, _FPAD), jnp.float32).at[:_NUM_ELEMENTS, :_FEAT_DIM].set(table)
    wp = jnp.zeros((_FPAD, _D_MODEL), jnp.float32).at[:_FEAT_DIM].set(W)
    return pl.pallas_call(
        _fuse_body,
        out_shape=jax.ShapeDtypeStruct((_VPAD, _D_MODEL), jnp.float32),
    )(tp, wp, b.reshape(1, _D_MODEL))


_mesh = plsc.VectorSubcoreMesh(
    core_axis_name="c", subcore_axis_name="s", num_cores=_NC, num_subcores=_NS
)


@functools.partial(
    pl.kernel,
    out_type=jax.ShapeDtypeStruct((_N_ATOMS, _D_MODEL), jnp.float32),
    mesh=_mesh,
    scratch_types=[
        pltpu.VMEM((_NBUF, _CHUNK), jnp.int32),
        pltpu.VMEM((_NBUF, _CHUNK, _D_MODEL), jnp.float32),
    ]
    + [pltpu.SemaphoreType.DMA] * (3 * _NBUF),
)
def _gather(x_hbm, fused_hbm, out_hbm, idx_v, rows_v, *sems):
    isems = sems[0:_NBUF]
    gsems = sems[_NBUF : 2 * _NBUF]
    wsems = sems[2 * _NBUF : 3 * _NBUF]
    wid = lax.axis_index("s") * _NC + lax.axis_index("c")

    def cid(i):
        return wid + i * _NW

    def start_idx(i, p):
        pltpu.async_copy(
            x_hbm.at[pl.ds(cid(i) * _CHUNK, _CHUNK)], idx_v.at[p], isems[p]
        )

    def start_gather(i, p):
        pltpu.make_async_copy(
            x_hbm.at[pl.ds(0, _CHUNK)], idx_v.at[p], isems[p]
        ).wait()
        pltpu.async_copy(fused_hbm.at[idx_v.at[p]], rows_v.at[p], gsems[p])

    def finish_chunk(i, p):
        pltpu.make_async_copy(
            fused_hbm.at[idx_v.at[p]], rows_v.at[p], gsems[p]
        ).wait()
        pltpu.async_copy(
            rows_v.at[p], out_hbm.at[pl.ds(cid(i) * _CHUNK, _CHUNK)], wsems[p]
        )

        @pl.when(cid(i + _NBUF) < _NUM_CHUNKS)
        def _prefetch():
            start_idx(i + _NBUF, p)

    for p in range(_NBUF):
        start_idx(p, p)

    def body(k, carry):
        for p in range(_NBUF):
            i = _NBUF * k + p
            p_prev = (p - 1) % _NBUF

            @pl.when(cid(i) < _NUM_CHUNKS)
            def _issue():
                @pl.when(k >= 1)
                def _drain_prev():
                    pltpu.make_async_copy(
                        rows_v.at[p], out_hbm.at[pl.ds(0, _CHUNK)], wsems[p]
                    ).wait()

                start_gather(i, p)

            if p == 0:
                @pl.when((k >= 1) & (cid(i - 1) < _NUM_CHUNKS))
                def _retire0():
                    finish_chunk(i - 1, p_prev)
            else:
                @pl.when(cid(i - 1) < _NUM_CHUNKS)
                def _retire():
                    finish_chunk(i - 1, p_prev)

        return carry

    lax.fori_loop(0, _NI // _NBUF, body, 0)

    @pl.when(cid(_NI - 1) < _NUM_CHUNKS)
    def _retire_last():
        finish_chunk(_NI - 1, (_NI - 1) % _NBUF)

    for p in range(_NBUF):
        pltpu.make_async_copy(
            rows_v.at[p], out_hbm.at[pl.ds(0, _CHUNK)], wsems[p]
        ).wait()


def kernel(x, table, W, b):
    fused = _fused_table(table, W, b)
    return _gather(x, fused)
